# baseline (speedup 1.0000x reference)
import functools

import jax
import jax.numpy as jnp
from jax import lax
from jax.experimental import pallas as pl
from jax.experimental.pallas import tpu as pltpu
from jax.experimental.pallas import tpu_sc as plsc

_B, _C, _H, _W = 8, 384, 32, 32
_NZ = 2
_HO = _H * (_NZ + 1) - _NZ
_WO = _W * (_NZ + 1) - _NZ

_WP = 128
_HOP, _WOP = 96, 128
_WOB = 96

_NW = 32
_W_PER_B = _NW // _B
_C_PER_W = _C // _W_PER_B
_UNIT_C = 4
_UNITS = _C_PER_W // _UNIT_C


@functools.partial(
    pl.kernel,
    out_type=jax.ShapeDtypeStruct((_B, _C, _HOP, _WOP), jnp.float32),
    mesh=plsc.VectorSubcoreMesh(core_axis_name="c", subcore_axis_name="s"),
    scratch_types=[
        pltpu.VMEM((2, _UNIT_C, _H, _W), jnp.float32),
        pltpu.VMEM((2, _UNIT_C, _HO, _WOB), jnp.float32),
        pltpu.SemaphoreType.DMA,
        pltpu.SemaphoreType.DMA,
        pltpu.SemaphoreType.DMA,
        pltpu.SemaphoreType.DMA,
    ],
    compiler_params=pltpu.CompilerParams(needs_layout_passes=False,
                                         use_tc_tiling_on_sc=False),
)
def _weave_sc(x_hbm, out_hbm, xbuf, obuf, sin0, sin1, sout0, sout1):
    nc = 2
    wid = lax.axis_index("s") * nc + lax.axis_index("c")
    b = wid // _W_PER_B
    c_base = (wid % _W_PER_B) * _C_PER_W
    sin = (sin0, sin1)
    sout = (sout0, sout1)

    iota = lax.iota(jnp.int32, 16)
    zeros16 = jnp.zeros((16,), jnp.float32)
    col_lo = iota * 3
    col_hi = col_lo + 48

    def _splat(v):
        return jnp.full((16,), v, jnp.int32)

    def _in_start(u, p):
        pltpu.async_copy(
            x_hbm.at[b, pl.ds(c_base + u * _UNIT_C, _UNIT_C),
                     pl.ds(0, _H), pl.ds(0, _W)],
            xbuf.at[p], sin[p])

    def _in_wait(p):
        pltpu.make_async_copy(
            x_hbm.at[0, pl.ds(0, _UNIT_C), pl.ds(0, _H), pl.ds(0, _W)],
            xbuf.at[p], sin[p]).wait()

    def _out_start(u, p):
        pltpu.async_copy(
            obuf.at[p],
            out_hbm.at[b, pl.ds(c_base + u * _UNIT_C, _UNIT_C),
                       pl.ds(0, _HO), pl.ds(0, _WOB)],
            sout[p])

    def _out_wait(p):
        pltpu.make_async_copy(
            obuf.at[p],
            out_hbm.at[0, pl.ds(0, _UNIT_C), pl.ds(0, _HO), pl.ds(0, _WOB)],
            sout[p]).wait()

    def _scatter(p):
        sp = _splat(p)
        for q in range(_UNIT_C):
            sq = _splat(q)
            for r in range(_H):
                sr = _splat(r)
                row_lo = plsc.load_gather(xbuf, [sp, sq, sr, iota])
                row_hi = plsc.load_gather(xbuf, [sp, sq, sr, iota + 16])
                dr = _splat(3 * r)
                plsc.store_scatter(obuf, [sp, sq, dr, col_lo], row_lo)
                plsc.store_scatter(obuf, [sp, sq, dr, col_hi], row_hi)

    _in_start(0, 0)
    _in_start(1, 1)

    def _zero(r, c):
        row = _splat(r)
        for p in range(2):
            for q in range(_UNIT_C):
                for o in (0, 16, 32, 48, 64, 80):
                    plsc.store_scatter(
                        obuf, [_splat(p), _splat(q), row, iota + o], zeros16)
        return c

    lax.fori_loop(0, _HO, _zero, 0)

    for u in (0, 1):
        p = u
        _in_wait(p)
        _scatter(p)
        _out_start(u, p)
        _in_start(u + 2, p)

    def _steady(i, c):
        for p in (0, 1):
            u = 2 * i + p
            _out_wait(p)
            _in_wait(p)
            _scatter(p)
            _out_start(u, p)
            _in_start(u + 2, p)
        return c

    lax.fori_loop(1, (_UNITS - 2) // 2, _steady, 0)

    for u in (_UNITS - 2, _UNITS - 1):
        p = u % 2
        _out_wait(p)
        _in_wait(p)
        _scatter(p)
        _out_start(u, p)
    _out_wait(0)
    _out_wait(1)


def kernel(x):
    pad_mat = jnp.eye(_W, _WP, dtype=jnp.float32)
    xp = lax.dot_general(x, pad_mat, (((3,), (0,)), ((), ())),
                         precision=lax.Precision.HIGHEST)
    padded = _weave_sc(xp)
    return padded[:, :, :_HO, :_WO]


# speedup vs baseline: 1.0388x; 1.0388x over previous
import functools

import jax
import jax.numpy as jnp
from jax import lax
from jax.experimental import pallas as pl
from jax.experimental.pallas import tpu as pltpu
from jax.experimental.pallas import tpu_sc as plsc

_B, _C, _H, _W = 8, 384, 32, 32
_NZ = 2
_HO = _H * (_NZ + 1) - _NZ
_WO = _W * (_NZ + 1) - _NZ

_WP = 128
_HOP, _WOP = 96, 128
_WOB = 96

_NW = 32
_W_PER_B = _NW // _B
_C_PER_W = _C // _W_PER_B
_UNIT_C = 4
_UNITS = _C_PER_W // _UNIT_C


@functools.partial(
    pl.kernel,
    out_type=jax.ShapeDtypeStruct((_B, _C, _HOP, _WOP), jnp.float32),
    mesh=plsc.VectorSubcoreMesh(core_axis_name="c", subcore_axis_name="s"),
    scratch_types=[
        pltpu.VMEM((2, _UNIT_C, _H, _W), jnp.float32),
        pltpu.VMEM((2, _UNIT_C, _HO, _WOB), jnp.float32),
        pltpu.SemaphoreType.DMA,
        pltpu.SemaphoreType.DMA,
        pltpu.SemaphoreType.DMA,
        pltpu.SemaphoreType.DMA,
    ],
    compiler_params=pltpu.CompilerParams(needs_layout_passes=False,
                                         use_tc_tiling_on_sc=False),
)
def _weave_sc(x_hbm, out_hbm, xbuf, obuf, sin0, sin1, sout0, sout1):
    nc = 2
    wid = lax.axis_index("s") * nc + lax.axis_index("c")
    b = wid // _W_PER_B
    c_base = (wid % _W_PER_B) * _C_PER_W
    sin = (sin0, sin1)
    sout = (sout0, sout1)

    iota = lax.iota(jnp.int32, 16)
    zeros16 = jnp.zeros((16,), jnp.float32)
    col_lo = iota * 3
    col_hi = col_lo + 48

    def _splat(v):
        return jnp.full((16,), v, jnp.int32)

    def _in_start(u, p):
        pltpu.async_copy(
            x_hbm.at[b, pl.ds(c_base + u * _UNIT_C, _UNIT_C),
                     pl.ds(0, _H), pl.ds(0, _W)],
            xbuf.at[p], sin[p])

    def _in_wait(p):
        pltpu.make_async_copy(
            x_hbm.at[0, pl.ds(0, _UNIT_C), pl.ds(0, _H), pl.ds(0, _W)],
            xbuf.at[p], sin[p]).wait()

    def _out_start(u, p):
        pltpu.async_copy(
            obuf.at[p],
            out_hbm.at[b, pl.ds(c_base + u * _UNIT_C, _UNIT_C),
                       pl.ds(0, _HO), pl.ds(0, _WOB)],
            sout[p])

    def _out_wait(p):
        pltpu.make_async_copy(
            obuf.at[p],
            out_hbm.at[0, pl.ds(0, _UNIT_C), pl.ds(0, _HO), pl.ds(0, _WOB)],
            sout[p]).wait()

    def _scatter(p):
        sp = _splat(p)
        for q in range(_UNIT_C):
            sq = _splat(q)
            for r in range(_H):
                sr = _splat(r)
                row_lo = plsc.load_gather(xbuf, [sp, sq, sr, iota])
                row_hi = plsc.load_gather(xbuf, [sp, sq, sr, iota + 16])
                dr = _splat(3 * r)
                plsc.store_scatter(obuf, [sp, sq, dr, col_lo], row_lo)
                plsc.store_scatter(obuf, [sp, sq, dr, col_hi], row_hi)

    _in_start(0, 0)
    _in_start(1, 1)

    def _zero(r, c):
        row = _splat(r)
        for p in range(2):
            for q in range(_UNIT_C):
                for o in (0, 16, 32, 48, 64, 80):
                    plsc.store_scatter(
                        obuf, [_splat(p), _splat(q), row, iota + o], zeros16)
        return c

    lax.fori_loop(0, _HO, _zero, 0)

    for u in (0, 1):
        p = u
        _in_wait(p)
        _scatter(p)
        _out_start(u, p)
        _in_start(u + 2, p)

    def _steady(i, c):
        for p in (0, 1):
            u = 2 * i + p
            _out_wait(p)
            _in_wait(p)
            _scatter(p)
            _out_start(u, p)
            _in_start(u + 2, p)
        return c

    lax.fori_loop(1, (_UNITS - 2) // 2, _steady, 0)

    for u in (_UNITS - 2, _UNITS - 1):
        p = u % 2
        _out_wait(p)
        _in_wait(p)
        _scatter(p)
        _out_start(u, p)
    _out_wait(0)
    _out_wait(1)


def kernel(x):
    xp = lax.dynamic_update_slice(
        jnp.zeros((_B, _C, _H, _WP), jnp.float32), x, (0, 0, 0, 0))
    padded = _weave_sc(xp)
    return padded[:, :, :_HO, :_WO]
